# initial kernel scaffold (unmeasured)
import jax
import jax.numpy as jnp
from jax import lax
from jax.experimental import pallas as pl
from jax.experimental.pallas import tpu as pltpu


def kernel(
    x,
):
    def body(*refs):
        pass

    out_shape = jax.ShapeDtypeStruct(..., jnp.float32)
    return pl.pallas_call(body, out_shape=out_shape)(...)


# baseline (device time: 2127641 ns/iter reference)
import jax
import jax.numpy as jnp
from jax import lax
from jax.experimental import pallas as pl
from jax.experimental.pallas import tpu as pltpu

M = 16384
N_OUT = 1024
M_OUT = 32768
HALF = M // 2


def kernel(x):
    def body(x_ref, out_ref, local_sem, sx_sem, rx_sem, sy_sem, ry_sem):
        my_x = lax.axis_index("x")
        my_y = lax.axis_index("y")
        o = 1 - my_x
        oy = 1 - my_y

        barrier = pltpu.get_barrier_semaphore()
        pl.semaphore_signal(barrier, inc=1, device_id=(o, my_y),
                            device_id_type=pltpu.DeviceIdType.MESH)
        pl.semaphore_signal(barrier, inc=1, device_id=(my_x, oy),
                            device_id_type=pltpu.DeviceIdType.MESH)
        pl.semaphore_wait(barrier, 2)

        local = pltpu.make_async_copy(
            x_ref.at[:, pl.ds(my_x * N_OUT, N_OUT)],
            out_ref.at[pl.ds(my_x * M, M), :],
            local_sem,
        )
        local.start()

        x_rdma = pltpu.make_async_remote_copy(
            src_ref=x_ref.at[pl.ds(my_y * HALF, HALF), pl.ds(o * N_OUT, N_OUT)],
            dst_ref=out_ref.at[pl.ds(my_x * M + my_y * HALF, HALF), :],
            send_sem=sx_sem,
            recv_sem=rx_sem,
            device_id=(o, my_y),
            device_id_type=pltpu.DeviceIdType.MESH,
        )
        x_rdma.start()
        x_rdma.wait_recv()

        y_rdma = pltpu.make_async_remote_copy(
            src_ref=out_ref.at[pl.ds(o * M + my_y * HALF, HALF), :],
            dst_ref=out_ref.at[pl.ds(o * M + my_y * HALF, HALF), :],
            send_sem=sy_sem,
            recv_sem=ry_sem,
            device_id=(my_x, oy),
            device_id_type=pltpu.DeviceIdType.MESH,
        )
        y_rdma.start()
        y_rdma.wait_recv()

        x_rdma.wait_send()
        y_rdma.wait_send()
        local.wait()

    return pl.pallas_call(
        body,
        out_shape=jax.ShapeDtypeStruct((M_OUT, N_OUT), jnp.float32),
        in_specs=[pl.BlockSpec(memory_space=pl.ANY)],
        out_specs=pl.BlockSpec(memory_space=pl.ANY),
        scratch_shapes=[
            pltpu.SemaphoreType.DMA,
            pltpu.SemaphoreType.DMA,
            pltpu.SemaphoreType.DMA,
            pltpu.SemaphoreType.DMA,
            pltpu.SemaphoreType.DMA,
        ],
        compiler_params=pltpu.CompilerParams(collective_id=0),
    )(x)


# device time: 502330 ns/iter; 4.2355x vs baseline; 4.2355x over previous
import jax
import jax.numpy as jnp
from jax import lax
from jax.experimental import pallas as pl
from jax.experimental.pallas import tpu as pltpu

M = 16384
N_OUT = 1024
M_OUT = 32768
HALF = M // 2

K = 16
R = HALF // K
LCH = 2048
LK = M // LCH


def kernel(x):
    def body(x_ref, out_ref, stage, lin_sems, lout_sems,
             sx_sems, rx_sems, sy_sems, ry_sems):
        my_x = lax.axis_index("x")
        my_y = lax.axis_index("y")
        o = 1 - my_x
        oy = 1 - my_y

        barrier = pltpu.get_barrier_semaphore()
        pl.semaphore_signal(barrier, inc=1, device_id=(o, my_y),
                            device_id_type=pl.DeviceIdType.MESH)
        pl.semaphore_signal(barrier, inc=1, device_id=(my_x, oy),
                            device_id_type=pl.DeviceIdType.MESH)
        pl.semaphore_wait(barrier, 2)

        x_rdmas = []
        for c in range(K):
            rd = pltpu.make_async_remote_copy(
                src_ref=x_ref.at[pl.ds(my_y * HALF + c * R, R),
                                 pl.ds(o * N_OUT, N_OUT)],
                dst_ref=out_ref.at[pl.ds(my_x * M + my_y * HALF + c * R, R), :],
                send_sem=sx_sems.at[c], recv_sem=rx_sems.at[c],
                device_id=(o, my_y), device_id_type=pl.DeviceIdType.MESH,
            )
            rd.start()
            x_rdmas.append(rd)

        pending = [None, None]
        for c in range(LK):
            slot = c % 2
            if pending[slot] is not None:
                pending[slot].wait()
            inc = pltpu.make_async_copy(
                x_ref.at[pl.ds(c * LCH, LCH), pl.ds(my_x * N_OUT, N_OUT)],
                stage.at[slot], lin_sems.at[slot])
            inc.start()
            inc.wait()
            outc = pltpu.make_async_copy(
                stage.at[slot],
                out_ref.at[pl.ds(my_x * M + c * LCH, LCH), :],
                lout_sems.at[slot])
            outc.start()
            pending[slot] = outc

        y_rdmas = []
        for c in range(K):
            x_rdmas[c].wait_recv()
            rd = pltpu.make_async_remote_copy(
                src_ref=out_ref.at[pl.ds(o * M + my_y * HALF + c * R, R), :],
                dst_ref=out_ref.at[pl.ds(o * M + my_y * HALF + c * R, R), :],
                send_sem=sy_sems.at[c], recv_sem=ry_sems.at[c],
                device_id=(my_x, oy), device_id_type=pl.DeviceIdType.MESH,
            )
            rd.start()
            y_rdmas.append(rd)

        for rd in y_rdmas:
            rd.wait_recv()
        for c in range(K):
            x_rdmas[c].wait_send()
            y_rdmas[c].wait_send()
        for p in pending:
            p.wait()

    return pl.pallas_call(
        body,
        out_shape=jax.ShapeDtypeStruct((M_OUT, N_OUT), jnp.float32),
        in_specs=[pl.BlockSpec(memory_space=pl.ANY)],
        out_specs=pl.BlockSpec(memory_space=pl.ANY),
        scratch_shapes=[
            pltpu.MemorySpace.VMEM((2, LCH, N_OUT), jnp.float32),
            pltpu.SemaphoreType.DMA((2,)),
            pltpu.SemaphoreType.DMA((2,)),
            pltpu.SemaphoreType.DMA((K,)),
            pltpu.SemaphoreType.DMA((K,)),
            pltpu.SemaphoreType.DMA((K,)),
            pltpu.SemaphoreType.DMA((K,)),
        ],
        compiler_params=pltpu.CompilerParams(collective_id=0),
    )(x)


# device time: 465386 ns/iter; 4.5718x vs baseline; 1.0794x over previous
import jax
import jax.numpy as jnp
from jax import lax
from jax.experimental import pallas as pl
from jax.experimental.pallas import tpu as pltpu

M = 16384
N_OUT = 1024
M_OUT = 32768
HALF = M // 2

K = 32
R = HALF // K
LCH = 2048
LK = M // LCH


def kernel(x):
    def body(x_ref, out_ref, stage, lin_sems, lout_sems,
             sx_sems, rx_sems, sy_sems, ry_sems):
        my_x = lax.axis_index("x")
        my_y = lax.axis_index("y")
        o = 1 - my_x
        oy = 1 - my_y

        barrier = pltpu.get_barrier_semaphore()
        pl.semaphore_signal(barrier, inc=1, device_id=(o, my_y),
                            device_id_type=pl.DeviceIdType.MESH)
        pl.semaphore_signal(barrier, inc=1, device_id=(my_x, oy),
                            device_id_type=pl.DeviceIdType.MESH)
        pl.semaphore_wait(barrier, 2)

        x_rdmas = []
        for c in range(K):
            rd = pltpu.make_async_remote_copy(
                src_ref=x_ref.at[pl.ds(my_y * HALF + c * R, R),
                                 pl.ds(o * N_OUT, N_OUT)],
                dst_ref=out_ref.at[pl.ds(my_x * M + my_y * HALF + c * R, R), :],
                send_sem=sx_sems.at[c], recv_sem=rx_sems.at[c],
                device_id=(o, my_y), device_id_type=pl.DeviceIdType.MESH,
            )
            rd.start()
            x_rdmas.append(rd)

        pending = [None, None]

        def local_step(c):
            slot = c % 2
            if pending[slot] is not None:
                pending[slot].wait()
            inc = pltpu.make_async_copy(
                x_ref.at[pl.ds(c * LCH, LCH), pl.ds(my_x * N_OUT, N_OUT)],
                stage.at[slot], lin_sems.at[slot])
            inc.start()
            inc.wait()
            outc = pltpu.make_async_copy(
                stage.at[slot],
                out_ref.at[pl.ds(my_x * M + c * LCH, LCH), :],
                lout_sems.at[slot])
            outc.start()
            pending[slot] = outc

        y_rdmas = []
        for c in range(K):
            x_rdmas[c].wait_recv()
            rd = pltpu.make_async_remote_copy(
                src_ref=out_ref.at[pl.ds(o * M + my_y * HALF + c * R, R), :],
                dst_ref=out_ref.at[pl.ds(o * M + my_y * HALF + c * R, R), :],
                send_sem=sy_sems.at[c], recv_sem=ry_sems.at[c],
                device_id=(my_x, oy), device_id_type=pl.DeviceIdType.MESH,
            )
            rd.start()
            y_rdmas.append(rd)
            if c % 2 == 1 and c // 2 < LK:
                local_step(c // 2)

        for rd in y_rdmas:
            rd.wait_recv()
        for c in range(K):
            x_rdmas[c].wait_send()
            y_rdmas[c].wait_send()
        for p in pending:
            p.wait()

    return pl.pallas_call(
        body,
        out_shape=jax.ShapeDtypeStruct((M_OUT, N_OUT), jnp.float32),
        in_specs=[pl.BlockSpec(memory_space=pl.ANY)],
        out_specs=pl.BlockSpec(memory_space=pl.ANY),
        scratch_shapes=[
            pltpu.MemorySpace.VMEM((2, LCH, N_OUT), jnp.float32),
            pltpu.SemaphoreType.DMA((2,)),
            pltpu.SemaphoreType.DMA((2,)),
            pltpu.SemaphoreType.DMA((K,)),
            pltpu.SemaphoreType.DMA((K,)),
            pltpu.SemaphoreType.DMA((K,)),
            pltpu.SemaphoreType.DMA((K,)),
        ],
        compiler_params=pltpu.CompilerParams(collective_id=0),
    )(x)


# device time: 461216 ns/iter; 4.6131x vs baseline; 1.0090x over previous
import jax
import jax.numpy as jnp
from jax import lax
from jax.experimental import pallas as pl
from jax.experimental.pallas import tpu as pltpu

M = 16384
N_OUT = 1024
M_OUT = 32768
HALF = M // 2

K = 64
R = HALF // K
LCH = 2048
LK = M // LCH


def kernel(x):
    def body(x_ref, out_ref, stage, lin_sems, lout_sems,
             sx_sems, rx_sems, sy_sems, ry_sems):
        my_x = lax.axis_index("x")
        my_y = lax.axis_index("y")
        o = 1 - my_x
        oy = 1 - my_y

        barrier = pltpu.get_barrier_semaphore()
        pl.semaphore_signal(barrier, inc=1, device_id=(o, my_y),
                            device_id_type=pl.DeviceIdType.MESH)
        pl.semaphore_signal(barrier, inc=1, device_id=(my_x, oy),
                            device_id_type=pl.DeviceIdType.MESH)
        pl.semaphore_wait(barrier, 2)

        x_rdmas = []
        for c in range(K):
            rd = pltpu.make_async_remote_copy(
                src_ref=x_ref.at[pl.ds(my_y * HALF + c * R, R),
                                 pl.ds(o * N_OUT, N_OUT)],
                dst_ref=out_ref.at[pl.ds(my_x * M + my_y * HALF + c * R, R), :],
                send_sem=sx_sems.at[c], recv_sem=rx_sems.at[c],
                device_id=(o, my_y), device_id_type=pl.DeviceIdType.MESH,
            )
            rd.start()
            x_rdmas.append(rd)

        pending = [None, None]

        def local_step(c):
            slot = c % 2
            if pending[slot] is not None:
                pending[slot].wait()
            inc = pltpu.make_async_copy(
                x_ref.at[pl.ds(c * LCH, LCH), pl.ds(my_x * N_OUT, N_OUT)],
                stage.at[slot], lin_sems.at[slot])
            inc.start()
            inc.wait()
            outc = pltpu.make_async_copy(
                stage.at[slot],
                out_ref.at[pl.ds(my_x * M + c * LCH, LCH), :],
                lout_sems.at[slot])
            outc.start()
            pending[slot] = outc

        y_rdmas = []
        for c in range(K):
            x_rdmas[c].wait_recv()
            rd = pltpu.make_async_remote_copy(
                src_ref=out_ref.at[pl.ds(o * M + my_y * HALF + c * R, R), :],
                dst_ref=out_ref.at[pl.ds(o * M + my_y * HALF + c * R, R), :],
                send_sem=sy_sems.at[c], recv_sem=ry_sems.at[c],
                device_id=(my_x, oy), device_id_type=pl.DeviceIdType.MESH,
            )
            rd.start()
            y_rdmas.append(rd)
            if c % 4 == 3 and c // 4 < LK:
                local_step(c // 4)

        for rd in y_rdmas:
            rd.wait_recv()
        for c in range(K):
            x_rdmas[c].wait_send()
            y_rdmas[c].wait_send()
        for p in pending:
            p.wait()

    return pl.pallas_call(
        body,
        out_shape=jax.ShapeDtypeStruct((M_OUT, N_OUT), jnp.float32),
        in_specs=[pl.BlockSpec(memory_space=pl.ANY)],
        out_specs=pl.BlockSpec(memory_space=pl.ANY),
        scratch_shapes=[
            pltpu.MemorySpace.VMEM((2, LCH, N_OUT), jnp.float32),
            pltpu.SemaphoreType.DMA((2,)),
            pltpu.SemaphoreType.DMA((2,)),
            pltpu.SemaphoreType.DMA((K,)),
            pltpu.SemaphoreType.DMA((K,)),
            pltpu.SemaphoreType.DMA((K,)),
            pltpu.SemaphoreType.DMA((K,)),
        ],
        compiler_params=pltpu.CompilerParams(collective_id=0),
    )(x)
